# Initial kernel scaffold; baseline (speedup 1.0000x reference)
import jax
import jax.numpy as jnp
from jax.experimental import pallas as pl


def kernel(x, v, g):
    raise NotImplementedError("write your pallas kernel here")


# speedup vs baseline: 2.8348x; 2.8348x over previous
import functools

import jax
import jax.numpy as jnp
from jax import lax
from jax.experimental import pallas as pl
from jax.experimental.pallas import tpu as pltpu
from jax.experimental.pallas import tpu_sc as plsc

_NUM_EMB = 1000
_EMB_DIM = 64
_B = 4096
_L = 200

_NW = 32
_B_PER_W = _B // _NW
_CH_STARTS = tuple(range(0, 192, 16)) + (184,)


def _prep_body(v_ref, g_ref, wT_ref):
    v = v_ref[...]
    s = jnp.sum(v * v, axis=1, keepdims=True)
    scale = g_ref[...] * lax.rsqrt(s)
    wT_ref[...] = (v * scale).T


def _prep(v, g):
    return pl.pallas_call(
        _prep_body,
        out_shape=jax.ShapeDtypeStruct((_EMB_DIM, _NUM_EMB), jnp.float32),
    )(v, g)


def _sc_embed_body(wT_hbm, x_hbm, out_hbm, wT_v, idx_v, stage_v, sem0, sem1):
    wid = lax.axis_index("s") * 2 + lax.axis_index("c")
    base = wid * _B_PER_W
    pltpu.sync_copy(wT_hbm, wT_v)
    pltpu.sync_copy(x_hbm.at[pl.ds(base, _B_PER_W)], idx_v)
    sems = (sem0, sem1)

    def gather_row(i, s):
        iv = [idx_v[i, pl.ds(st, 16)] for st in _CH_STARTS]

        def d_body(d, carry):
            dv = jnp.full((16,), d, jnp.int32)
            for c, st in enumerate(_CH_STARTS):
                stage_v[s, d, pl.ds(st, 16)] = plsc.load_gather(wT_v, [dv, iv[c]])
            return carry

        lax.fori_loop(0, _EMB_DIM, d_body, 0)

    def pair_body(ip, carry):
        for s in range(2):
            i = ip * 2 + s

            @pl.when(ip > 0)
            def _wait():
                pltpu.make_async_copy(
                    stage_v.at[s], out_hbm.at[base + i - 2], sems[s]
                ).wait()

            gather_row(i, s)
            pltpu.async_copy(stage_v.at[s], out_hbm.at[base + i], sems[s])
        return carry

    lax.fori_loop(0, _B_PER_W // 2, pair_body, 0)
    pltpu.make_async_copy(
        stage_v.at[0], out_hbm.at[base + _B_PER_W - 2], sem0
    ).wait()
    pltpu.make_async_copy(
        stage_v.at[1], out_hbm.at[base + _B_PER_W - 1], sem1
    ).wait()


@functools.cache
def _build_sc_embed():
    return pl.kernel(
        _sc_embed_body,
        out_type=jax.ShapeDtypeStruct((_B, _EMB_DIM, _L), jnp.float32),
        mesh=plsc.VectorSubcoreMesh(core_axis_name="c", subcore_axis_name="s"),
        scratch_types=[
            pltpu.VMEM((_EMB_DIM, _NUM_EMB), jnp.float32),
            pltpu.VMEM((_B_PER_W, _L), jnp.int32),
            pltpu.VMEM((2, _EMB_DIM, _L), jnp.float32),
            pltpu.SemaphoreType.DMA,
            pltpu.SemaphoreType.DMA,
        ],
        compiler_params=pltpu.CompilerParams(
            use_tc_tiling_on_sc=False, needs_layout_passes=False
        ),
    )


def kernel(x, v, g):
    wT = _prep(v, g)
    return _build_sc_embed()(wT, x.astype(jnp.int32))
